# Initial kernel scaffold; baseline (speedup 1.0000x reference)
import jax
import jax.numpy as jnp
from jax.experimental import pallas as pl


def kernel(x, var_emb):
    raise NotImplementedError("write your pallas kernel here")


# speedup vs baseline: 7.8854x; 7.8854x over previous
import jax
import jax.numpy as jnp
from jax.experimental import pallas as pl

_BLOCK_BL = 128


def _bcast_kernel(emb_ref, out_ref):
    out_ref[...] = jnp.broadcast_to(emb_ref[...][None], out_ref.shape)


def kernel(x, var_emb):
    B, L, D = x.shape
    d_model = var_emb.shape[1]
    BL = B * L
    emb = var_emb[:D]

    out = pl.pallas_call(
        _bcast_kernel,
        grid=(BL // _BLOCK_BL,),
        in_specs=[pl.BlockSpec((D, d_model), lambda i: (0, 0))],
        out_specs=pl.BlockSpec((_BLOCK_BL, D, d_model), lambda i: (i, 0, 0)),
        out_shape=jax.ShapeDtypeStruct((BL, D, d_model), var_emb.dtype),
    )(emb)
    return out.reshape(B, L, D, d_model)
